# baseline (device time: 207325 ns/iter reference)
import jax
import jax.numpy as jnp
from jax import lax
from jax.experimental import pallas as pl
from jax.experimental.pallas import tpu as pltpu

N_DEV = 16
M = 64
D = 1024
H = 2048
G = 4


def kernel(x, Win0, Wout0, Win1, Wout1, Win2, Wout2):
    xb = x.astype(jnp.bfloat16)
    win0, wout0, win1, wout1, win2, wout2 = (
        w.astype(jnp.bfloat16) for w in (Win0, Wout0, Win1, Wout1, Win2, Wout2)
    )

    def body(x_ref, win0_r, wout0_r, win1_r, wout1_r, win2_r, wout2_r, out_ref,
             xfull, pacc, pb, rs_recv,
             ag_send_sems, ag_recv_sems, rs_send_sems, rs_recv_sems):
        me = lax.axis_index("i")
        left = lax.rem(me + N_DEV - 1, N_DEV)

        barrier = pltpu.get_barrier_semaphore()
        for k in range(1, N_DEV):
            pl.semaphore_signal(barrier, inc=1,
                                device_id=(lax.rem(me + k, N_DEV),),
                                device_id_type=pl.DeviceIdType.MESH)
        pl.semaphore_wait(barrier, N_DEV - 1)

        layers = ((win0_r, wout0_r), (win1_r, wout1_r), (win2_r, wout2_r))
        for l, (win, wout) in enumerate(layers):
            if l == 0:
                xfull[0] = x_ref[...]

            ag_sends = []
            for k in range(1, N_DEV):
                snd = pltpu.make_async_remote_copy(
                    src_ref=xfull.at[0], dst_ref=xfull.at[k],
                    send_sem=ag_send_sems.at[k - 1],
                    recv_sem=ag_recv_sems.at[k - 1],
                    device_id=(lax.rem(me + k, N_DEV),),
                    device_id_type=pl.DeviceIdType.MESH,
                )
                snd.start()
                ag_sends.append(snd)
            for snd in ag_sends:
                snd.wait_send()

            for k in range(1, N_DEV):
                rcv = pltpu.make_async_remote_copy(
                    src_ref=xfull.at[k], dst_ref=xfull.at[k],
                    send_sem=ag_send_sems.at[k - 1],
                    recv_sem=ag_recv_sems.at[k - 1],
                    device_id=(left,), device_id_type=pl.DeviceIdType.MESH,
                )
                rcv.wait_recv()
            for g in range(N_DEV // G):
                xg = xfull[pl.ds(g * G, G)].reshape(G * M, D)
                hg = jnp.maximum(
                    jnp.dot(xg, win[...], preferred_element_type=jnp.float32),
                    0.0,
                ).astype(jnp.bfloat16)
                pg = jnp.dot(hg, wout[...], preferred_element_type=jnp.float32)
                pacc[pl.ds(g * G, G)] = pg.reshape(G, M, D)
                pb[pl.ds(g * G, G)] = pg.astype(jnp.bfloat16).reshape(G, M, D)

            rs_sends = []
            for s in range(1, N_DEV):
                snd = pltpu.make_async_remote_copy(
                    src_ref=pb.at[s], dst_ref=rs_recv.at[s - 1],
                    send_sem=rs_send_sems.at[s - 1],
                    recv_sem=rs_recv_sems.at[s - 1],
                    device_id=(lax.rem(me - s + N_DEV, N_DEV),),
                    device_id_type=pl.DeviceIdType.MESH,
                )
                snd.start()
                rs_sends.append(snd)
            for s in range(1, N_DEV):
                rcv = pltpu.make_async_remote_copy(
                    src_ref=rs_recv.at[s - 1], dst_ref=rs_recv.at[s - 1],
                    send_sem=rs_send_sems.at[s - 1],
                    recv_sem=rs_recv_sems.at[s - 1],
                    device_id=(left,), device_id_type=pl.DeviceIdType.MESH,
                )
                rcv.wait_recv()
            for snd in rs_sends:
                snd.wait_send()

            y = pacc[0]
            for s in range(1, N_DEV):
                y = y + rs_recv[s - 1].astype(jnp.float32)
            if l < 2:
                xfull[0] = y.astype(jnp.bfloat16)
            else:
                out_ref[...] = y

    return pl.pallas_call(
        body,
        out_shape=jax.ShapeDtypeStruct((M, D), jnp.float32),
        in_specs=[pl.BlockSpec(memory_space=pltpu.VMEM)] * 7,
        out_specs=pl.BlockSpec(memory_space=pltpu.VMEM),
        scratch_shapes=[
            pltpu.VMEM((N_DEV, M, D), jnp.bfloat16),
            pltpu.VMEM((N_DEV, M, D), jnp.float32),
            pltpu.VMEM((N_DEV, M, D), jnp.bfloat16),
            pltpu.VMEM((N_DEV - 1, M, D), jnp.bfloat16),
            pltpu.SemaphoreType.DMA((N_DEV - 1,)),
            pltpu.SemaphoreType.DMA((N_DEV - 1,)),
            pltpu.SemaphoreType.DMA((N_DEV - 1,)),
            pltpu.SemaphoreType.DMA((N_DEV - 1,)),
        ],
        compiler_params=pltpu.CompilerParams(collective_id=0),
    )(xb, win0, wout0, win1, wout1, win2, wout2)


# device time: 207061 ns/iter; 1.0013x vs baseline; 1.0013x over previous
import jax
import jax.numpy as jnp
from jax import lax
from jax.experimental import pallas as pl
from jax.experimental.pallas import tpu as pltpu

N_DEV = 16
M = 64
D = 1024
H = 2048
G = 4


def kernel(x, Win0, Wout0, Win1, Wout1, Win2, Wout2):
    xb = x.astype(jnp.bfloat16)
    win0, wout0, win1, wout1, win2, wout2 = (
        w.astype(jnp.bfloat16) for w in (Win0, Wout0, Win1, Wout1, Win2, Wout2)
    )

    def body(x_ref, win0_r, wout0_r, win1_r, wout1_r, win2_r, wout2_r, out_ref,
             xfull, pacc, pb, rs_recv,
             ag_send_sems, ag_recv_sems, rs_send_sems, rs_recv_sems):
        me = lax.axis_index("i")
        left = lax.rem(me + N_DEV - 1, N_DEV)

        barrier = pltpu.get_barrier_semaphore()
        for k in range(1, N_DEV):
            pl.semaphore_signal(barrier, inc=1,
                                device_id=(lax.rem(me + k, N_DEV),),
                                device_id_type=pl.DeviceIdType.MESH)
        pl.semaphore_wait(barrier, N_DEV - 1)

        layers = ((win0_r, wout0_r), (win1_r, wout1_r), (win2_r, wout2_r))
        for l, (win, wout) in enumerate(layers):
            if l == 0:
                xfull[0] = x_ref[...]

            ag_sends = []
            for k in range(1, N_DEV):
                snd = pltpu.make_async_remote_copy(
                    src_ref=xfull.at[0], dst_ref=xfull.at[k],
                    send_sem=ag_send_sems.at[k - 1],
                    recv_sem=ag_recv_sems.at[k - 1],
                    device_id=(lax.rem(me + k, N_DEV),),
                    device_id_type=pl.DeviceIdType.MESH,
                )
                snd.start()
                ag_sends.append(snd)
            for snd in ag_sends:
                snd.wait_send()

            for k in range(1, N_DEV):
                rcv = pltpu.make_async_remote_copy(
                    src_ref=xfull.at[k], dst_ref=xfull.at[k],
                    send_sem=ag_send_sems.at[k - 1],
                    recv_sem=ag_recv_sems.at[k - 1],
                    device_id=(left,), device_id_type=pl.DeviceIdType.MESH,
                )
                rcv.wait_recv()
            for g in range(N_DEV // G):
                xg = xfull[pl.ds(g * G, G)].reshape(G * M, D)
                hg = jnp.maximum(
                    jnp.dot(xg, win[...], preferred_element_type=jnp.float32),
                    0.0,
                ).astype(jnp.bfloat16)
                pg = jnp.dot(hg, wout[...], preferred_element_type=jnp.float32)
                pacc[pl.ds(g * G, G)] = pg.reshape(G, M, D)
                pb[pl.ds(g * G, G)] = pg.astype(jnp.bfloat16).reshape(G, M, D)

            rs_sends = []
            for s in range(1, N_DEV):
                snd = pltpu.make_async_remote_copy(
                    src_ref=pb.at[s], dst_ref=rs_recv.at[s - 1],
                    send_sem=rs_send_sems.at[s - 1],
                    recv_sem=rs_recv_sems.at[s - 1],
                    device_id=(lax.rem(me - s + N_DEV, N_DEV),),
                    device_id_type=pl.DeviceIdType.MESH,
                )
                snd.start()
                rs_sends.append(snd)
            y = pacc[0]
            for s in range(1, N_DEV):
                rcv = pltpu.make_async_remote_copy(
                    src_ref=rs_recv.at[s - 1], dst_ref=rs_recv.at[s - 1],
                    send_sem=rs_send_sems.at[s - 1],
                    recv_sem=rs_recv_sems.at[s - 1],
                    device_id=(left,), device_id_type=pl.DeviceIdType.MESH,
                )
                rcv.wait_recv()
                y = y + rs_recv[s - 1].astype(jnp.float32)
            for snd in rs_sends:
                snd.wait_send()
            if l < 2:
                xfull[0] = y.astype(jnp.bfloat16)
            else:
                out_ref[...] = y

    return pl.pallas_call(
        body,
        out_shape=jax.ShapeDtypeStruct((M, D), jnp.float32),
        in_specs=[pl.BlockSpec(memory_space=pltpu.VMEM)] * 7,
        out_specs=pl.BlockSpec(memory_space=pltpu.VMEM),
        scratch_shapes=[
            pltpu.VMEM((N_DEV, M, D), jnp.bfloat16),
            pltpu.VMEM((N_DEV, M, D), jnp.float32),
            pltpu.VMEM((N_DEV, M, D), jnp.bfloat16),
            pltpu.VMEM((N_DEV - 1, M, D), jnp.bfloat16),
            pltpu.SemaphoreType.DMA((N_DEV - 1,)),
            pltpu.SemaphoreType.DMA((N_DEV - 1,)),
            pltpu.SemaphoreType.DMA((N_DEV - 1,)),
            pltpu.SemaphoreType.DMA((N_DEV - 1,)),
        ],
        compiler_params=pltpu.CompilerParams(collective_id=0),
    )(xb, win0, wout0, win1, wout1, win2, wout2)


# device time: 196735 ns/iter; 1.0538x vs baseline; 1.0525x over previous
import jax
import jax.numpy as jnp
from jax import lax
from jax.experimental import pallas as pl
from jax.experimental.pallas import tpu as pltpu

N_DEV = 16
M = 64
D = 1024
H = 2048
G = 4


def kernel(x, Win0, Wout0, Win1, Wout1, Win2, Wout2):
    xb = x.astype(jnp.bfloat16)
    win0, wout0, win1, wout1, win2, wout2 = (
        w.astype(jnp.bfloat16) for w in (Win0, Wout0, Win1, Wout1, Win2, Wout2)
    )

    def body(x_ref, win0_r, wout0_r, win1_r, wout1_r, win2_r, wout2_r, out_ref,
             xfull, pacc, pb, rs_recv,
             ag_send_sems, ag_recv_sems, rs_send_sems, rs_recv_sems):
        me = lax.axis_index("i")
        left = lax.rem(me + N_DEV - 1, N_DEV)

        barrier = pltpu.get_barrier_semaphore()
        for k in range(1, N_DEV):
            pl.semaphore_signal(barrier, inc=1,
                                device_id=(lax.rem(me + k, N_DEV),),
                                device_id_type=pl.DeviceIdType.MESH)
        pl.semaphore_wait(barrier, N_DEV - 1)

        layers = ((win0_r, wout0_r), (win1_r, wout1_r), (win2_r, wout2_r))
        for l, (win, wout) in enumerate(layers):
            if l == 0:
                xfull[0] = x_ref[...]

            ag_sends = []
            for k in range(1, N_DEV):
                snd = pltpu.make_async_remote_copy(
                    src_ref=xfull.at[0], dst_ref=xfull.at[k],
                    send_sem=ag_send_sems.at[k - 1],
                    recv_sem=ag_recv_sems.at[k - 1],
                    device_id=(lax.rem(me + k, N_DEV),),
                    device_id_type=pl.DeviceIdType.MESH,
                )
                snd.start()
                ag_sends.append(snd)
            for snd in ag_sends:
                snd.wait_send()

            for k in range(1, N_DEV):
                rcv = pltpu.make_async_remote_copy(
                    src_ref=xfull.at[k], dst_ref=xfull.at[k],
                    send_sem=ag_send_sems.at[k - 1],
                    recv_sem=ag_recv_sems.at[k - 1],
                    device_id=(left,), device_id_type=pl.DeviceIdType.MESH,
                )
                rcv.wait_recv()
            rs_sends = []
            for g in range(N_DEV // G):
                xg = xfull[pl.ds(g * G, G)].reshape(G * M, D)
                hg = jnp.maximum(
                    jnp.dot(xg, win[...], preferred_element_type=jnp.float32),
                    0.0,
                ).astype(jnp.bfloat16)
                pg = jnp.dot(hg, wout[...], preferred_element_type=jnp.float32)
                pacc[pl.ds(g * G, G)] = pg.reshape(G, M, D)
                pb[pl.ds(g * G, G)] = pg.astype(jnp.bfloat16).reshape(G, M, D)
                for s in range(max(1, g * G), (g + 1) * G):
                    snd = pltpu.make_async_remote_copy(
                        src_ref=pb.at[s], dst_ref=rs_recv.at[s - 1],
                        send_sem=rs_send_sems.at[s - 1],
                        recv_sem=rs_recv_sems.at[s - 1],
                        device_id=(lax.rem(me - s + N_DEV, N_DEV),),
                        device_id_type=pl.DeviceIdType.MESH,
                    )
                    snd.start()
                    rs_sends.append(snd)
            y = pacc[0]
            for s in range(1, N_DEV):
                rcv = pltpu.make_async_remote_copy(
                    src_ref=rs_recv.at[s - 1], dst_ref=rs_recv.at[s - 1],
                    send_sem=rs_send_sems.at[s - 1],
                    recv_sem=rs_recv_sems.at[s - 1],
                    device_id=(left,), device_id_type=pl.DeviceIdType.MESH,
                )
                rcv.wait_recv()
                y = y + rs_recv[s - 1].astype(jnp.float32)
            for snd in rs_sends:
                snd.wait_send()
            if l < 2:
                xfull[0] = y.astype(jnp.bfloat16)
            else:
                out_ref[...] = y

    return pl.pallas_call(
        body,
        out_shape=jax.ShapeDtypeStruct((M, D), jnp.float32),
        in_specs=[pl.BlockSpec(memory_space=pltpu.VMEM)] * 7,
        out_specs=pl.BlockSpec(memory_space=pltpu.VMEM),
        scratch_shapes=[
            pltpu.VMEM((N_DEV, M, D), jnp.bfloat16),
            pltpu.VMEM((N_DEV, M, D), jnp.float32),
            pltpu.VMEM((N_DEV, M, D), jnp.bfloat16),
            pltpu.VMEM((N_DEV - 1, M, D), jnp.bfloat16),
            pltpu.SemaphoreType.DMA((N_DEV - 1,)),
            pltpu.SemaphoreType.DMA((N_DEV - 1,)),
            pltpu.SemaphoreType.DMA((N_DEV - 1,)),
            pltpu.SemaphoreType.DMA((N_DEV - 1,)),
        ],
        compiler_params=pltpu.CompilerParams(collective_id=0),
    )(xb, win0, wout0, win1, wout1, win2, wout2)
